# baseline (speedup 1.0000x reference)
import functools

import jax
import jax.numpy as jnp
from jax import lax
from jax.experimental import pallas as pl
from jax.experimental.pallas import tpu as pltpu
from jax.experimental.pallas import tpu_sc as plsc


def _iota16():
    return lax.iota(jnp.int32, 16)


@functools.lru_cache(maxsize=None)
def _transpose_call(v, d):
    info = plsc.get_sparse_core_info()
    nw = info.num_cores * info.num_subcores
    vf = (v // 128) * 128
    nfull = vf // 128
    vrem = v - vf
    per_w = (nfull + nw - 1) // nw

    mesh = plsc.VectorSubcoreMesh(core_axis_name="c", subcore_axis_name="s")

    def shuffle_window(src_ref, dst_ref, r_loc):
        for p in range(2):
            col = 2 * r_loc + p
            for g in range(4):
                vals = plsc.load_gather(
                    src_ref, [_iota16() + g * 16, jnp.full((16,), col, jnp.int32)])
                dst_ref[r_loc, pl.ds(p * d + g * 16, 16)] = vals

    def body(tt_hbm, rem_hbm, s_hbm, w_v, sb_v, rem_v, gs0, gs1, ws0, ws1):
        cid = lax.axis_index("c")
        sid = lax.axis_index("s")
        wid = sid * info.num_cores + cid
        gsem = (gs0, gs1)
        wsem = (ws0, ws1)

        def stage(i, slot):
            w = wid + i * nw

            @pl.when(w < nfull)
            def _():
                off = pl.multiple_of(w * 128, 128)
                pltpu.async_copy(tt_hbm.at[:, pl.ds(off, 128)],
                                 w_v.at[slot], gsem[slot])

        def wait_stage(slot):
            pltpu.make_async_copy(tt_hbm.at[:, pl.ds(0, 128)],
                                  w_v.at[slot], gsem[slot]).wait()

        def wait_write(slot):
            pltpu.make_async_copy(sb_v.at[slot], s_hbm.at[pl.ds(0, 64), :],
                                  wsem[slot]).wait()

        stage(0, 0)
        stage(1, 1)

        ntrip = per_w // 2 + 1

        def trip(t, carry):
            for k in range(2):
                i = 2 * t + k
                w = wid + i * nw

                @pl.when(w < nfull)
                def _(i=i, k=k, w=w):
                    wait_stage(k)

                    def rows(r, c2):
                        shuffle_window(w_v.at[k], sb_v.at[k], r)
                        return c2

                    lax.fori_loop(0, 64, rows, 0)

                    @pl.when(i >= 2)
                    def _():
                        wait_write(k)

                    off = pl.multiple_of(w * 64, 8)
                    pltpu.async_copy(sb_v.at[k], s_hbm.at[pl.ds(off, 64), :],
                                     wsem[k])
                    stage(i + 2, k)
            return carry

        lax.fori_loop(0, ntrip, trip, 0)

        wait_write(0)
        wait_write(1)

        @pl.when(wid == nw - 1)
        def _():
            pltpu.sync_copy(rem_hbm, rem_v)

            def rrows(r, carry):
                for p in range(2):
                    j = 2 * r + p
                    for g in range(4):
                        vals = plsc.load_gather(
                            rem_v, [(_iota16() + g * 16) * vrem + j])
                        sb_v[0, r, pl.ds(p * d + g * 16, 16)] = vals
                return carry

            lax.fori_loop(0, vrem // 2, rrows, 0)
            pltpu.sync_copy(sb_v.at[0, pl.ds(0, vrem // 2), :],
                            s_hbm.at[pl.ds(vf // 2, vrem // 2), :])

    return pl.kernel(
        body,
        out_type=jax.ShapeDtypeStruct((v // 2, 2 * d), jnp.float32),
        mesh=mesh,
        scratch_types=(
            pltpu.VMEM((2, d, 128), jnp.float32),
            pltpu.VMEM((2, 64, 2 * d), jnp.float32),
            pltpu.VMEM((d * vrem,), jnp.float32),
            pltpu.SemaphoreType.DMA, pltpu.SemaphoreType.DMA,
            pltpu.SemaphoreType.DMA, pltpu.SemaphoreType.DMA,
        ),
        compiler_params=pltpu.CompilerParams(use_tc_tiling_on_sc=True, needs_layout_passes=False),
    )


@functools.lru_cache(maxsize=None)
def _gather_call(s, b, v, d):
    info = plsc.get_sparse_core_info()
    nw = info.num_cores * info.num_subcores
    bc = b // nw

    mesh = plsc.VectorSubcoreMesh(core_axis_name="c", subcore_axis_name="s")

    def body(s_hbm, ids_hbm, out_hbm, idx_v, q_v, o_v, g_v, a_v,
             gs0, gs1, ws0, ws1):
        cid = lax.axis_index("c")
        sid = lax.axis_index("s")
        wid = sid * info.num_cores + cid
        b_off = pl.multiple_of(wid * bc, 128)
        gsem = (gs0, gs1)
        wsem = (ws0, ws1)
        pltpu.sync_copy(ids_hbm.at[:, pl.ds(b_off, bc)], idx_v)

        def prep_and_fire(si, slot):
            for g in range(bc // 16):
                raw = idx_v[si, pl.ds(g * 16, 16)]
                q_v[slot, pl.ds(g * 16, 16)] = lax.shift_right_logical(raw, 1)
                o_v[slot, pl.ds(g * 16, 16)] = lax.shift_left(
                    lax.bitwise_and(raw, 1), 6)
            pltpu.async_copy(s_hbm.at[q_v.at[slot]], g_v.at[slot], gsem[slot])

        def wait_gather(slot):
            pltpu.make_async_copy(s_hbm.at[q_v.at[slot]], g_v.at[slot],
                                  gsem[slot]).wait()

        def wait_write(slot):
            pltpu.make_async_copy(a_v.at[slot], out_hbm.at[0, :, pl.ds(0, bc)],
                                  wsem[slot]).wait()

        prep_and_fire(0, 0)

        ntrip = s // 2

        def trip(t, carry):
            for k in range(2):
                si = 2 * t + k

                @pl.when(si + 1 < s)
                def _(si=si, k=k):
                    prep_and_fire(si + 1, 1 - k)
                wait_gather(k)

                def crow(c, carry2):
                    for g in range(bc // 16):
                        col = o_v[k, pl.ds(g * 16, 16)] + c
                        vals = plsc.load_gather(
                            g_v, [jnp.full((16,), k, jnp.int32),
                                  _iota16() + g * 16, col])
                        a_v[k, c, pl.ds(g * 16, 16)] = vals
                    return carry2

                lax.fori_loop(0, d, crow, 0)

                @pl.when(si >= 2)
                def _(k=k):
                    wait_write(k)
                pltpu.async_copy(a_v.at[k], out_hbm.at[si, :, pl.ds(b_off, bc)],
                                 wsem[k])
            return carry

        lax.fori_loop(0, ntrip, trip, 0)
        wait_write(0)
        wait_write(1)

    return pl.kernel(
        body,
        out_type=jax.ShapeDtypeStruct((s, d, b), jnp.float32),
        mesh=mesh,
        scratch_types=(
            pltpu.VMEM((s, bc), jnp.int32),
            pltpu.VMEM((2, bc), jnp.int32),
            pltpu.VMEM((2, bc), jnp.int32),
            pltpu.VMEM((2, bc, 2 * d), jnp.float32),
            pltpu.VMEM((2, d, bc), jnp.float32),
            pltpu.SemaphoreType.DMA, pltpu.SemaphoreType.DMA,
            pltpu.SemaphoreType.DMA, pltpu.SemaphoreType.DMA,
        ),
        compiler_params=pltpu.CompilerParams(use_tc_tiling_on_sc=True, needs_layout_passes=False),
    )


def kernel(input_ids, table):
    s, b = input_ids.shape
    v, d = table.shape
    tt = table.T
    vf = (v // 128) * 128
    rem = tt[:, vf:].reshape(-1)
    pair_s = _transpose_call(v, d)(tt, rem)
    out_t = _gather_call(s, b, v, d)(pair_s, input_ids)
    return out_t.transpose(0, 2, 1)


# speedup vs baseline: 1.2460x; 1.2460x over previous
import functools

import jax
import jax.numpy as jnp
from jax import lax
from jax.experimental import pallas as pl
from jax.experimental.pallas import tpu as pltpu
from jax.experimental.pallas import tpu_sc as plsc

_WV = 256


def _iota16():
    return lax.iota(jnp.int32, 16)


@functools.lru_cache(maxsize=None)
def _transpose_call(v, d):
    info = plsc.get_sparse_core_info()
    nw = info.num_cores * info.num_subcores
    vf = (v // _WV) * _WV
    nfull = vf // _WV
    vrem = v - vf
    per_w = (nfull + nw - 1) // nw
    sr = _WV // 2

    mesh = plsc.VectorSubcoreMesh(core_axis_name="c", subcore_axis_name="s")

    def body(tt_hbm, rem_hbm, s_hbm, w_v, sb_v, rem_v, gs0, gs1, ws0, ws1):
        cid = lax.axis_index("c")
        sid = lax.axis_index("s")
        wid = sid * info.num_cores + cid
        gsem = (gs0, gs1)
        wsem = (ws0, ws1)
        rowvecs = [_iota16() + g * 16 for g in range(d // 16)]

        def stage(i, slot):
            w = wid + i * nw

            @pl.when(w < nfull)
            def _():
                off = pl.multiple_of(w * _WV, 128)
                pltpu.async_copy(tt_hbm.at[:, pl.ds(off, _WV)],
                                 w_v.at[slot], gsem[slot])

        def wait_stage(slot):
            pltpu.make_async_copy(tt_hbm.at[:, pl.ds(0, _WV)],
                                  w_v.at[slot], gsem[slot]).wait()

        def wait_write(slot):
            pltpu.make_async_copy(sb_v.at[slot], s_hbm.at[pl.ds(0, sr), :],
                                  wsem[slot]).wait()

        stage(0, 0)
        stage(1, 1)

        ntrip = per_w // 2 + 1

        def trip(t, carry):
            for k in range(2):
                i = 2 * t + k
                w = wid + i * nw

                @pl.when(w < nfull)
                def _(i=i, k=k, w=w):
                    wait_stage(k)

                    def rows(r8, c2):
                        r0 = r8 * 8
                        for dr in range(8):
                            r = r0 + dr
                            for p in range(2):
                                col = jnp.full((16,), 2 * r + p, jnp.int32)
                                for g in range(d // 16):
                                    vals = plsc.load_gather(
                                        w_v.at[k], [rowvecs[g], col])
                                    sb_v[k, r, pl.ds(p * d + g * 16, 16)] = vals
                        return c2

                    lax.fori_loop(0, sr // 8, rows, 0)

                    @pl.when(i >= 2)
                    def _():
                        wait_write(k)

                    off = pl.multiple_of(w * sr, 8)
                    pltpu.async_copy(sb_v.at[k], s_hbm.at[pl.ds(off, sr), :],
                                     wsem[k])
                    stage(i + 2, k)
            return carry

        lax.fori_loop(0, ntrip, trip, 0)

        wait_write(0)
        wait_write(1)

        @pl.when(wid == nw - 1)
        def _():
            pltpu.sync_copy(rem_hbm, rem_v)

            def rrows(r, carry):
                for p in range(2):
                    j = 2 * r + p
                    for g in range(d // 16):
                        vals = plsc.load_gather(
                            rem_v, [(_iota16() + g * 16) * vrem + j])
                        sb_v[0, r, pl.ds(p * d + g * 16, 16)] = vals
                return carry

            lax.fori_loop(0, vrem // 2, rrows, 0)
            pltpu.sync_copy(sb_v.at[0, pl.ds(0, vrem // 2), :],
                            s_hbm.at[pl.ds(vf // 2, vrem // 2), :])

    return pl.kernel(
        body,
        out_type=jax.ShapeDtypeStruct((v // 2, 2 * d), jnp.float32),
        mesh=mesh,
        scratch_types=(
            pltpu.VMEM((2, d, _WV), jnp.float32),
            pltpu.VMEM((2, sr, 2 * d), jnp.float32),
            pltpu.VMEM((d * vrem,), jnp.float32),
            pltpu.SemaphoreType.DMA, pltpu.SemaphoreType.DMA,
            pltpu.SemaphoreType.DMA, pltpu.SemaphoreType.DMA,
        ),
        compiler_params=pltpu.CompilerParams(
            use_tc_tiling_on_sc=True, needs_layout_passes=False),
    )


@functools.lru_cache(maxsize=None)
def _gather_call(s, b, v, d):
    info = plsc.get_sparse_core_info()
    nw = info.num_cores * info.num_subcores
    bc = b // nw
    nch = s // 2

    mesh = plsc.VectorSubcoreMesh(core_axis_name="c", subcore_axis_name="s")

    def body(s_hbm, ids_hbm, out_hbm, idx_v, q_v, o_v, g_v, a_v,
             gs0, gs1, ws0, ws1):
        cid = lax.axis_index("c")
        sid = lax.axis_index("s")
        wid = sid * info.num_cores + cid
        b_off = pl.multiple_of(wid * bc, 128)
        gsem = (gs0, gs1)
        wsem = (ws0, ws1)
        pltpu.sync_copy(ids_hbm.at[:, pl.ds(b_off, bc)], idx_v)
        rowvecs = [_iota16() + g * 16 for g in range(bc // 16)]

        def prep_and_fire(ci, slot):
            for h in range(2):
                si = 2 * ci + h
                for g in range(bc // 16):
                    raw = idx_v[si, pl.ds(g * 16, 16)]
                    q_v[slot, h, pl.ds(g * 16, 16)] = (
                        lax.shift_right_logical(raw, 1))
                    o_v[slot, h, pl.ds(g * 16, 16)] = lax.shift_left(
                        lax.bitwise_and(raw, 1), 6)
                pltpu.async_copy(s_hbm.at[q_v.at[slot, h]],
                                 g_v.at[slot, h], gsem[slot])

        def wait_gather(slot):
            for h in range(2):
                pltpu.make_async_copy(s_hbm.at[q_v.at[slot, h]],
                                      g_v.at[slot, h], gsem[slot]).wait()

        def wait_write(slot):
            pltpu.make_async_copy(
                a_v.at[slot], out_hbm.at[pl.ds(0, 2), :, pl.ds(0, bc)],
                wsem[slot]).wait()

        prep_and_fire(0, 0)

        ntrip = nch // 2

        def trip(t, carry):
            for k in range(2):
                ci = 2 * t + k

                @pl.when(ci + 1 < nch)
                def _(ci=ci, k=k):
                    prep_and_fire(ci + 1, 1 - k)
                wait_gather(k)

                offs = [[o_v[k, h, pl.ds(g * 16, 16)] for g in range(bc // 16)]
                        for h in range(2)]

                def crow(c4, c2):
                    c0 = c4 * 4
                    for dc in range(4):
                        c = c0 + dc
                        for h in range(2):
                            for g in range(bc // 16):
                                vals = plsc.load_gather(
                                    g_v.at[k, h],
                                    [rowvecs[g], offs[h][g] + c])
                                a_v[k, h, c, pl.ds(g * 16, 16)] = vals
                    return c2

                lax.fori_loop(0, d // 4, crow, 0)

                @pl.when(ci >= 2)
                def _(k=k):
                    wait_write(k)
                s_off = pl.multiple_of(2 * ci, 2)
                pltpu.async_copy(
                    a_v.at[k], out_hbm.at[pl.ds(s_off, 2), :,
                                          pl.ds(b_off, bc)], wsem[k])
            return carry

        lax.fori_loop(0, ntrip, trip, 0)
        wait_write(0)
        wait_write(1)

    return pl.kernel(
        body,
        out_type=jax.ShapeDtypeStruct((s, d, b), jnp.float32),
        mesh=mesh,
        scratch_types=(
            pltpu.VMEM((s, bc), jnp.int32),
            pltpu.VMEM((2, 2, bc), jnp.int32),
            pltpu.VMEM((2, 2, bc), jnp.int32),
            pltpu.VMEM((2, 2, bc, 2 * d), jnp.float32),
            pltpu.VMEM((2, 2, d, bc), jnp.float32),
            pltpu.SemaphoreType.DMA, pltpu.SemaphoreType.DMA,
            pltpu.SemaphoreType.DMA, pltpu.SemaphoreType.DMA,
        ),
        compiler_params=pltpu.CompilerParams(
            use_tc_tiling_on_sc=True, needs_layout_passes=False),
    )


def kernel(input_ids, table):
    s, b = input_ids.shape
    v, d = table.shape
    tt = table.T
    vf = (v // _WV) * _WV
    rem = tt[:, vf:].reshape(-1)
    pair_s = _transpose_call(v, d)(tt, rem)
    out_t = _gather_call(s, b, v, d)(pair_s, input_ids)
    return out_t.transpose(0, 2, 1)


# speedup vs baseline: 3.4979x; 2.8072x over previous
import functools

import jax
import jax.numpy as jnp
from jax import lax
from jax.experimental import pallas as pl
from jax.experimental.pallas import tpu as pltpu
from jax.experimental.pallas import tpu_sc as plsc


@functools.lru_cache(maxsize=None)
def _gather_call(s, b, v, d):
    info = plsc.get_sparse_core_info()
    nw = info.num_cores * info.num_subcores
    bc = b // nw
    nch = s // 2

    mesh = plsc.VectorSubcoreMesh(core_axis_name="c", subcore_axis_name="s")

    def body(table_hbm, ids_hbm, out_hbm, idx_v, g_v, gs0, gs1, ws0, ws1):
        cid = lax.axis_index("c")
        sid = lax.axis_index("s")
        wid = sid * info.num_cores + cid
        b_off = pl.multiple_of(wid * bc, 128)
        gsem = (gs0, gs1)
        wsem = (ws0, ws1)
        pltpu.sync_copy(ids_hbm.at[:, pl.ds(b_off, bc)], idx_v)

        def fire(ci, slot):
            for h in range(2):
                pltpu.async_copy(table_hbm.at[idx_v.at[2 * ci + h]],
                                 g_v.at[slot, h], gsem[slot])

        def wait_gather(slot):
            for h in range(2):
                pltpu.make_async_copy(table_hbm.at[idx_v.at[0]],
                                      g_v.at[slot, h], gsem[slot]).wait()

        def wait_write(slot):
            pltpu.make_async_copy(
                g_v.at[slot],
                out_hbm.at[pl.ds(0, 2), pl.ds(0, bc), :], wsem[slot]).wait()

        fire(0, 0)

        ntrip = nch // 2

        def trip(t, carry):
            for k in range(2):
                ci = 2 * t + k

                @pl.when(ci + 1 < nch)
                def _(ci=ci, k=k):
                    @pl.when(ci >= 1)
                    def _():
                        wait_write(1 - k)
                    fire(ci + 1, 1 - k)
                wait_gather(k)
                s_off = pl.multiple_of(2 * ci, 2)
                pltpu.async_copy(
                    g_v.at[k],
                    out_hbm.at[pl.ds(s_off, 2), pl.ds(b_off, bc), :], wsem[k])
            return carry

        lax.fori_loop(0, ntrip, trip, 0)
        wait_write(0)
        wait_write(1)

    return pl.kernel(
        body,
        out_type=jax.ShapeDtypeStruct((s, b, 2 * d), jnp.float32),
        mesh=mesh,
        scratch_types=(
            pltpu.VMEM((s, bc), jnp.int32),
            pltpu.VMEM((2, 2, bc, 2 * d), jnp.float32),
            pltpu.SemaphoreType.DMA, pltpu.SemaphoreType.DMA,
            pltpu.SemaphoreType.DMA, pltpu.SemaphoreType.DMA,
        ),
        compiler_params=pltpu.CompilerParams(
            use_tc_tiling_on_sc=True, needs_layout_passes=False),
    )


def kernel(input_ids, table):
    s, b = input_ids.shape
    v, d = table.shape
    padded = jnp.pad(table, ((0, 0), (0, 128 - d)))
    out = _gather_call(s, b, v, d)(padded, input_ids)
    return out[:, :, :d]
